# baseline (speedup 1.0000x reference)
import functools

import jax
import jax.numpy as jnp
from jax import lax
from jax.experimental import pallas as pl
from jax.experimental.pallas import tpu as pltpu
from jax.experimental.pallas import tpu_sc as plsc

N = 10000
E = 320000
D = 128

NC = 2
NS = 16
NW = NC * NS

BLK = 40
NBLK0 = 252
NBLK1 = 248
P = 6
RQ = 8
IQ = 12
N_ACC = N
ROWS_PER_TILE = 632


def _msg_kernel(x_ref, w_ref, b_ref, o_ref):
    acc = jnp.dot(x_ref[...], w_ref[...],
                  preferred_element_type=jnp.float32,
                  precision=lax.Precision.DEFAULT)
    o_ref[...] = jnp.maximum(acc + b_ref[...], 0.0)


def _out_kernel(p_ref, w_ref, bo_ref, o_ref):
    s = p_ref[0] + p_ref[1]
    acc = jnp.dot(s, w_ref[...],
                  preferred_element_type=jnp.float32,
                  precision=lax.Precision.DEFAULT)
    o_ref[...] = acc + bo_ref[...]


def _segment_sum_sc(h, src_blk, dst_blk, zeros):
    mesh = plsc.VectorSubcoreMesh(core_axis_name="c", subcore_axis_name="s")

    @functools.partial(
        pl.kernel,
        mesh=mesh,
        out_type=jax.ShapeDtypeStruct((NC, N_ACC, D), jnp.float32),
        scratch_types=[
            pltpu.VMEM((IQ, BLK), jnp.int32),
            pltpu.VMEM((IQ, BLK), jnp.int32),
            pltpu.VMEM((RQ, BLK, D), jnp.float32),
            pltpu.VMEM_SHARED((N_ACC, D), jnp.float32),
            pltpu.SemaphoreType.DMA,
            pltpu.SemaphoreType.DMA,
            pltpu.SemaphoreType.DMA,
        ],
    )
    def segsum(h_hbm, src_hbm, dst_hbm, zero_hbm, out_hbm,
               sidx, didx, rows, acc, isem, gsem, ssem):
        c = lax.axis_index("c")
        s = lax.axis_index("s")

        nblk = lax.select(c == 0, NBLK0, NBLK1)
        base = lax.select(c == 0, s * NBLK0, NBLK0 * NS + s * NBLK1)

        def idx_start(b, q):
            pltpu.async_copy(src_hbm.at[base + b], sidx.at[q], isem)
            pltpu.async_copy(dst_hbm.at[base + b], didx.at[q], isem)

        def idx_wait(b, q):
            pltpu.make_async_copy(src_hbm.at[base + b], sidx.at[q],
                                  isem).wait()
            pltpu.make_async_copy(dst_hbm.at[base + b], didx.at[q],
                                  isem).wait()

        def gather_start(q, j):
            pltpu.async_copy(h_hbm.at[sidx.at[q]], rows.at[j], gsem)

        def gather_wait(q, j):
            pltpu.make_async_copy(h_hbm.at[sidx.at[q]], rows.at[j],
                                  gsem).wait()

        def scatter_start(q, j):
            pltpu.async_copy(rows.at[j], acc.at[didx.at[q]], ssem, add=True)

        def scatter_wait(q, j):
            pltpu.make_async_copy(rows.at[j], acc.at[didx.at[q]],
                                  ssem).wait()

        @pl.loop(0, P + 1)
        def _(k):
            idx_start(k, lax.rem(k, IQ))

        @pl.loop(0, P)
        def _(k):
            idx_wait(k, lax.rem(k, IQ))
            gather_start(lax.rem(k, IQ), lax.rem(k, RQ))

        r0 = lax.min(s * ROWS_PER_TILE, N_ACC - ROWS_PER_TILE)
        pltpu.sync_copy(zero_hbm, acc.at[pl.ds(r0, ROWS_PER_TILE)])
        plsc.subcore_barrier()

        @pl.loop(0, NBLK0)
        def _(b):
            @pl.when(b < nblk)
            def _():
                jb = lax.rem(b, RQ)

                @pl.when(b + P + 1 < nblk)
                def _():
                    idx_start(b + P + 1, lax.rem(b + P + 1, IQ))

                gather_wait(lax.rem(b, IQ), jb)

                @pl.when(b >= 2)
                def _():
                    scatter_wait(lax.rem(b - 2, IQ), lax.rem(b - 2, RQ))

                scatter_start(lax.rem(b, IQ), jb)

                @pl.when(b + P < nblk)
                def _():
                    qn = lax.rem(b + P, IQ)
                    idx_wait(b + P, qn)
                    gather_start(qn, lax.rem(b + P, RQ))

        @pl.when(c == 0)
        def _():
            scatter_wait(lax.rem(NBLK0 - 2, IQ), (NBLK0 - 2) % RQ)
            scatter_wait(lax.rem(NBLK0 - 1, IQ), (NBLK0 - 1) % RQ)

        @pl.when(c == 1)
        def _():
            scatter_wait(lax.rem(NBLK1 - 2, IQ), (NBLK1 - 2) % RQ)
            scatter_wait(lax.rem(NBLK1 - 1, IQ), (NBLK1 - 1) % RQ)

        plsc.subcore_barrier()
        pltpu.sync_copy(acc.at[pl.ds(r0, ROWS_PER_TILE)],
                        out_hbm.at[c, pl.ds(r0, ROWS_PER_TILE)])

    return segsum(h, src_blk, dst_blk, zeros)


def kernel(x, edge_index, Wm, bm, Wo, bo):
    h = pl.pallas_call(
        _msg_kernel,
        out_shape=jax.ShapeDtypeStruct((N, D), jnp.float32),
    )(x, Wm, bm.reshape(1, D))

    zeros = jnp.zeros((ROWS_PER_TILE, D), jnp.float32)
    src_blk = edge_index[0].reshape(E // BLK, BLK)
    dst_blk = edge_index[1].reshape(E // BLK, BLK)
    parts = _segment_sum_sc(h, src_blk, dst_blk, zeros)

    out = pl.pallas_call(
        _out_kernel,
        out_shape=jax.ShapeDtypeStruct((N, D), jnp.float32),
    )(parts, Wo, bo.reshape(1, D))
    return out


# speedup vs baseline: 1.0886x; 1.0886x over previous
import functools

import jax
import jax.numpy as jnp
from jax import lax
from jax.experimental import pallas as pl
from jax.experimental.pallas import tpu as pltpu
from jax.experimental.pallas import tpu_sc as plsc

N = 10000
E = 320000
D = 128

NC = 2
NS = 16
NW = NC * NS

BLK = 80
NBLK0 = 127
NBLK1 = 123
P = 3
RQ = 4
IQ = 8
N_ACC = N
ROWS_PER_TILE = 632


def _msg_kernel(x_ref, w_ref, b_ref, o_ref):
    acc = jnp.dot(x_ref[...], w_ref[...],
                  preferred_element_type=jnp.float32,
                  precision=lax.Precision.DEFAULT)
    o_ref[...] = jnp.maximum(acc + b_ref[...], 0.0)


def _out_kernel(p_ref, w_ref, bo_ref, o_ref):
    s = p_ref[0] + p_ref[1]
    acc = jnp.dot(s, w_ref[...],
                  preferred_element_type=jnp.float32,
                  precision=lax.Precision.DEFAULT)
    o_ref[...] = acc + bo_ref[...]


def _segment_sum_sc(h, src_blk, dst_blk, zeros):
    mesh = plsc.VectorSubcoreMesh(core_axis_name="c", subcore_axis_name="s")

    @functools.partial(
        pl.kernel,
        mesh=mesh,
        out_type=jax.ShapeDtypeStruct((NC, N_ACC, D), jnp.float32),
        scratch_types=[
            pltpu.VMEM((IQ, BLK), jnp.int32),
            pltpu.VMEM((IQ, BLK), jnp.int32),
            pltpu.VMEM((RQ, BLK, D), jnp.float32),
            pltpu.VMEM_SHARED((N_ACC, D), jnp.float32),
            pltpu.SemaphoreType.DMA,
            pltpu.SemaphoreType.DMA,
            pltpu.SemaphoreType.DMA,
        ],
    )
    def segsum(h_hbm, src_hbm, dst_hbm, zero_hbm, out_hbm,
               sidx, didx, rows, acc, isem, gsem, ssem):
        c = lax.axis_index("c")
        s = lax.axis_index("s")

        nblk = lax.select(c == 0, NBLK0, NBLK1)
        base = lax.select(c == 0, s * NBLK0, NBLK0 * NS + s * NBLK1)

        def idx_start(b, q):
            pltpu.async_copy(src_hbm.at[base + b], sidx.at[q], isem)
            pltpu.async_copy(dst_hbm.at[base + b], didx.at[q], isem)

        def idx_wait(b, q):
            pltpu.make_async_copy(src_hbm.at[base + b], sidx.at[q],
                                  isem).wait()
            pltpu.make_async_copy(dst_hbm.at[base + b], didx.at[q],
                                  isem).wait()

        def gather_start(q, j):
            pltpu.async_copy(h_hbm.at[sidx.at[q]], rows.at[j], gsem)

        def gather_wait(q, j):
            pltpu.make_async_copy(h_hbm.at[sidx.at[q]], rows.at[j],
                                  gsem).wait()

        def scatter_start(q, j):
            pltpu.async_copy(rows.at[j], acc.at[didx.at[q]], ssem, add=True)

        def scatter_wait(q, j):
            pltpu.make_async_copy(rows.at[j], acc.at[didx.at[q]],
                                  ssem).wait()

        @pl.loop(0, P + 1)
        def _(k):
            idx_start(k, lax.rem(k, IQ))

        @pl.loop(0, P)
        def _(k):
            idx_wait(k, lax.rem(k, IQ))
            gather_start(lax.rem(k, IQ), lax.rem(k, RQ))

        r0 = lax.min(s * ROWS_PER_TILE, N_ACC - ROWS_PER_TILE)
        pltpu.sync_copy(zero_hbm.at[pl.ds(r0, ROWS_PER_TILE)],
                        acc.at[pl.ds(r0, ROWS_PER_TILE)])
        plsc.subcore_barrier()

        @pl.loop(0, NBLK0)
        def _(b):
            @pl.when(b < nblk)
            def _():
                jb = lax.rem(b, RQ)

                @pl.when(b + P + 1 < nblk)
                def _():
                    idx_start(b + P + 1, lax.rem(b + P + 1, IQ))

                gather_wait(lax.rem(b, IQ), jb)

                @pl.when(b >= 1)
                def _():
                    scatter_wait(lax.rem(b - 1, IQ), lax.rem(b - 1, RQ))

                scatter_start(lax.rem(b, IQ), jb)

                @pl.when(b + P < nblk)
                def _():
                    qn = lax.rem(b + P, IQ)
                    idx_wait(b + P, qn)
                    gather_start(qn, lax.rem(b + P, RQ))

        @pl.when(c == 0)
        def _():
            scatter_wait(lax.rem(NBLK0 - 1, IQ), (NBLK0 - 1) % RQ)

        @pl.when(c == 1)
        def _():
            scatter_wait(lax.rem(NBLK1 - 1, IQ), (NBLK1 - 1) % RQ)

        plsc.subcore_barrier()
        pltpu.sync_copy(acc.at[pl.ds(r0, ROWS_PER_TILE)],
                        out_hbm.at[c, pl.ds(r0, ROWS_PER_TILE)])

    return segsum(h, src_blk, dst_blk, zeros)


def kernel(x, edge_index, Wm, bm, Wo, bo):
    h = pl.pallas_call(
        _msg_kernel,
        out_shape=jax.ShapeDtypeStruct((N, D), jnp.float32),
    )(x, Wm, bm.reshape(1, D))

    zeros = jnp.zeros((N_ACC, D), jnp.float32)
    src_blk = edge_index[0].reshape(E // BLK, BLK)
    dst_blk = edge_index[1].reshape(E // BLK, BLK)
    parts = _segment_sum_sc(h, src_blk, dst_blk, zeros)

    out = pl.pallas_call(
        _out_kernel,
        out_shape=jax.ShapeDtypeStruct((N, D), jnp.float32),
    )(parts, Wo, bo.reshape(1, D))
    return out
